# baseline (speedup 1.0000x reference)
import functools

import jax
import jax.numpy as jnp
import numpy as np
from jax import lax
from jax.experimental import pallas as pl
from jax.experimental.pallas import tpu as pltpu
from jax.experimental.pallas import tpu_sc as plsc

VOCAB = 100000
MAX_LEN = 200
D_MODEL = 64
BATCH = 4096

NC, NS = 2, 16
NW = NC * NS
BT = BATCH // NW
NBUF = 5
SNB = 2
DT = D_MODEL // 8
NCH = D_MODEL // 16


@functools.partial(
    pl.kernel,
    out_type=jax.ShapeDtypeStruct((MAX_LEN, DT, NW, 8, BT), jnp.float32),
    mesh=plsc.VectorSubcoreMesh(core_axis_name="c", subcore_axis_name="s"),
    scratch_types=[
        pltpu.VMEM((MAX_LEN, BT), jnp.int32),
        pltpu.VMEM((MAX_LEN, D_MODEL), jnp.float32),
        pltpu.VMEM((NBUF, BT, D_MODEL), jnp.float32),
        pltpu.VMEM((SNB, DT, 8, BT), jnp.float32),
        pltpu.SemaphoreType.DMA((NBUF,)),
        pltpu.SemaphoreType.DMA((SNB,)),
    ],
    compiler_params=pltpu.CompilerParams(
        use_tc_tiling_on_sc=False, needs_layout_passes=False),
)
def _sc_embed(tok_hbm, table_hbm, pe_hbm, out_hbm,
              idx_v, pe_v, gbuf, sbuf, gsem, ssem):
    wid = lax.axis_index("s") * NC + lax.axis_index("c")

    pltpu.sync_copy(tok_hbm.at[wid], idx_v)
    pltpu.sync_copy(pe_hbm, pe_v)

    def fire_gather(l, b):
        pltpu.async_copy(table_hbm.at[idx_v.at[l]], gbuf.at[b], gsem.at[b])

    def wait_gather(b):
        pltpu.make_async_copy(
            table_hbm.at[idx_v.at[0]], gbuf.at[b], gsem.at[b]).wait()

    for b in range(NBUF):
        fire_gather(b, b)

    iota = lax.iota(jnp.int32, 16)
    blv = [iota + (q * 16) for q in range(BT // 16)]

    @pl.loop(0, MAX_LEN // NBUF)
    def _outer(o):
        for b in range(NBUF):
            l = o * NBUF + b
            sb = b % SNB
            wait_gather(b)
            if b >= SNB:
                pltpu.make_async_copy(
                    sbuf.at[sb], out_hbm.at[0, :, 0], ssem.at[sb]).wait()
            else:
                @pl.when(o > 0)
                def _():
                    pltpu.make_async_copy(
                        sbuf.at[sb], out_hbm.at[0, :, 0], ssem.at[sb]).wait()

            l_vec = lax.broadcast(l, (16,))

            @pl.loop(0, NCH)
            def _chunk(c):
                c16 = lax.broadcast(c * 16, (16,))
                c2 = lax.broadcast(c * 2, (16,))
                for t in range(16):
                    m = (iota + t) & 15
                    col = m + c16
                    pe_vec = plsc.load_gather(pe_v, [l_vec, col])
                    dtv = (m >> 3) + c2
                    dsv = m & 7
                    for q in range(BT // 16):
                        v = plsc.load_gather(gbuf.at[b], [blv[q], col])
                        plsc.store_scatter(
                            sbuf.at[sb], [dtv, dsv, blv[q]], v + pe_vec)

            @pl.when(o < MAX_LEN // NBUF - 1)
            def _():
                fire_gather(l + NBUF, b)

            pltpu.async_copy(sbuf.at[sb], out_hbm.at[l, :, wid], ssem.at[sb])

    for sb in range(SNB):
        pltpu.make_async_copy(
            sbuf.at[sb], out_hbm.at[0, :, 0], ssem.at[sb]).wait()


def _positional_encoding() -> np.ndarray:
    pos = np.arange(MAX_LEN, dtype=np.float32)[:, None]
    i = np.arange(D_MODEL // 2, dtype=np.float32)[None, :]
    denom = np.power(10000.0, (2.0 * i) / D_MODEL)
    pe = np.zeros((MAX_LEN, D_MODEL), dtype=np.float32)
    pe[:, 0::2] = np.sin(pos / denom)
    pe[:, 1::2] = np.cos(pos / denom)
    return pe


def kernel(tokens, table):
    pe = jnp.asarray(_positional_encoding())
    tokT = tokens.reshape(NW, BT, MAX_LEN).transpose(0, 2, 1)
    out5 = _sc_embed(tokT, table, pe)
    out = out5.transpose(2, 4, 0, 1, 3).reshape(BATCH, MAX_LEN, D_MODEL)
    return out


# speedup vs baseline: 1.0089x; 1.0089x over previous
import functools

import jax
import jax.numpy as jnp
import numpy as np
from jax import lax
from jax.experimental import pallas as pl
from jax.experimental.pallas import tpu as pltpu
from jax.experimental.pallas import tpu_sc as plsc

VOCAB = 100000
MAX_LEN = 200
D_MODEL = 64
BATCH = 4096

NC, NS = 2, 16
NW = NC * NS
BT = BATCH // NW
NBUF = 4
SNB = 2
DT = D_MODEL // 8
NCH = D_MODEL // 16


@functools.partial(
    pl.kernel,
    out_type=jax.ShapeDtypeStruct((MAX_LEN, DT, NW, 8, BT), jnp.float32),
    mesh=plsc.VectorSubcoreMesh(core_axis_name="c", subcore_axis_name="s"),
    scratch_types=[
        pltpu.VMEM((MAX_LEN, BT), jnp.int32),
        pltpu.VMEM((MAX_LEN, D_MODEL), jnp.float32),
        pltpu.VMEM((NBUF, BT, D_MODEL), jnp.float32),
        pltpu.VMEM((SNB, DT, 8, BT), jnp.float32),
        pltpu.SemaphoreType.DMA((NBUF,)),
        pltpu.SemaphoreType.DMA((SNB,)),
    ],
    compiler_params=pltpu.CompilerParams(
        use_tc_tiling_on_sc=False, needs_layout_passes=False),
)
def _sc_embed(tok_hbm, table_hbm, pe_hbm, out_hbm,
              idx_v, pe_v, gbuf, sbuf, gsem, ssem):
    wid = lax.axis_index("s") * NC + lax.axis_index("c")

    pltpu.sync_copy(tok_hbm.at[wid], idx_v)
    pltpu.sync_copy(pe_hbm, pe_v)

    def fire_gather(l, b):
        pltpu.async_copy(table_hbm.at[idx_v.at[l]], gbuf.at[b], gsem.at[b])

    def wait_gather(b):
        pltpu.make_async_copy(
            table_hbm.at[idx_v.at[0]], gbuf.at[b], gsem.at[b]).wait()

    for b in range(NBUF):
        fire_gather(b, b)

    iota = lax.iota(jnp.int32, 16)
    blv = [iota + (q * 16) for q in range(BT // 16)]

    @pl.loop(0, MAX_LEN // NBUF)
    def _outer(o):
        for b in range(NBUF):
            l = o * NBUF + b
            sb = b % SNB
            wait_gather(b)
            if b >= SNB:
                pltpu.make_async_copy(
                    sbuf.at[sb], out_hbm.at[0, :, 0], ssem.at[sb]).wait()
            else:
                @pl.when(o > 0)
                def _():
                    pltpu.make_async_copy(
                        sbuf.at[sb], out_hbm.at[0, :, 0], ssem.at[sb]).wait()

            l_vec = lax.broadcast(l, (16,))

            @pl.loop(0, NCH)
            def _chunk(c):
                c16 = lax.broadcast(c * 16, (16,))
                c2 = lax.broadcast(c * 2, (16,))
                for t in range(16):
                    m = (iota + t) & 15
                    col = m + c16
                    pe_vec = plsc.load_gather(pe_v, [l_vec, col])
                    dtv = (m >> 3) + c2
                    dsv = m & 7
                    for q in range(BT // 16):
                        v = plsc.load_gather(gbuf.at[b], [blv[q], col])
                        plsc.store_scatter(
                            sbuf.at[sb], [dtv, dsv, blv[q]], v + pe_vec)

            pltpu.async_copy(sbuf.at[sb], out_hbm.at[l, :, wid], ssem.at[sb])

            @pl.when(o < MAX_LEN // NBUF - 1)
            def _():
                fire_gather(l + NBUF, b)

    for sb in range(SNB):
        pltpu.make_async_copy(
            sbuf.at[sb], out_hbm.at[0, :, 0], ssem.at[sb]).wait()


def _positional_encoding() -> np.ndarray:
    pos = np.arange(MAX_LEN, dtype=np.float32)[:, None]
    i = np.arange(D_MODEL // 2, dtype=np.float32)[None, :]
    denom = np.power(10000.0, (2.0 * i) / D_MODEL)
    pe = np.zeros((MAX_LEN, D_MODEL), dtype=np.float32)
    pe[:, 0::2] = np.sin(pos / denom)
    pe[:, 1::2] = np.cos(pos / denom)
    return pe


def kernel(tokens, table):
    pe = jnp.asarray(_positional_encoding())
    tokT = tokens.reshape(NW, BT, MAX_LEN).transpose(0, 2, 1)
    out5 = _sc_embed(tokT, table, pe)
    out = out5.transpose(2, 4, 0, 1, 3).reshape(BATCH, MAX_LEN, D_MODEL)
    return out


# speedup vs baseline: 1.8297x; 1.8137x over previous
import functools

import jax
import jax.numpy as jnp
import numpy as np
from jax import lax
from jax.experimental import pallas as pl
from jax.experimental.pallas import tpu as pltpu
from jax.experimental.pallas import tpu_sc as plsc

VOCAB = 100000
MAX_LEN = 200
D_MODEL = 64
BATCH = 4096

NC, NS = 2, 16
NW = NC * NS
BT = BATCH // NW
NBUF = 4
SNB = 2
DT = D_MODEL // 8
NCH = D_MODEL // 16


@functools.partial(
    pl.kernel,
    out_type=jax.ShapeDtypeStruct((MAX_LEN, DT, NW, 8, BT), jnp.float32),
    mesh=plsc.VectorSubcoreMesh(core_axis_name="c", subcore_axis_name="s"),
    scratch_types=[
        pltpu.VMEM((MAX_LEN, BT), jnp.int32),
        pltpu.VMEM((MAX_LEN, D_MODEL), jnp.float32),
        pltpu.VMEM((NBUF, BT, D_MODEL), jnp.float32),
        pltpu.VMEM((SNB, DT, 8, BT), jnp.float32),
        pltpu.SemaphoreType.DMA((NBUF,)),
        pltpu.SemaphoreType.DMA((SNB,)),
    ],
    compiler_params=pltpu.CompilerParams(
        use_tc_tiling_on_sc=False, needs_layout_passes=False),
)
def _sc_embed(tok_hbm, table_hbm, pe_hbm, out_hbm,
              idx_v, pe_v, gbuf, sbuf, gsem, ssem):
    wid = lax.axis_index("s") * NC + lax.axis_index("c")

    pltpu.sync_copy(tok_hbm.at[wid], idx_v)
    pltpu.sync_copy(pe_hbm, pe_v)

    def fire_gather(l, b):
        pltpu.async_copy(table_hbm.at[idx_v.at[l]], gbuf.at[b], gsem.at[b])

    def wait_gather(b):
        pltpu.make_async_copy(
            table_hbm.at[idx_v.at[0]], gbuf.at[b], gsem.at[b]).wait()

    for b in range(NBUF):
        fire_gather(b, b)

    iota = lax.iota(jnp.int32, 16)
    blv = [iota + (q * 16) for q in range(BT // 16)]

    @pl.loop(0, MAX_LEN // NBUF)
    def _outer(o):
        for b in range(NBUF):
            l = o * NBUF + b
            sb = b % SNB
            wait_gather(b)
            if b >= SNB:
                pltpu.make_async_copy(
                    sbuf.at[sb], out_hbm.at[0, :, 0], ssem.at[sb]).wait()
            else:
                @pl.when(o > 0)
                def _():
                    pltpu.make_async_copy(
                        sbuf.at[sb], out_hbm.at[0, :, 0], ssem.at[sb]).wait()

            l_vec = lax.broadcast(l, (16,))

            @pl.loop(0, NCH)
            def _chunk(c):
                c16 = lax.broadcast(c * 16, (16,))
                c2 = lax.broadcast(c * 2, (16,))
                for t in range(16):
                    m = (iota + t) & 15
                    col = m + c16
                    pe_vec = plsc.load_gather(pe_v, [l_vec, col])
                    dtv = (m >> 3) + c2
                    dsv = m & 7
                    vs = [plsc.load_gather(gbuf.at[b], [blv[q], col])
                          for q in range(BT // 16)]
                    for q in range(BT // 16):
                        plsc.store_scatter(
                            sbuf.at[sb], [dtv, dsv, blv[q]], vs[q] + pe_vec)

            pltpu.async_copy(sbuf.at[sb], out_hbm.at[l, :, wid], ssem.at[sb])

            @pl.when(o < MAX_LEN // NBUF - 1)
            def _():
                fire_gather(l + NBUF, b)

    for sb in range(SNB):
        pltpu.make_async_copy(
            sbuf.at[sb], out_hbm.at[0, :, 0], ssem.at[sb]).wait()


def _positional_encoding() -> np.ndarray:
    pos = np.arange(MAX_LEN, dtype=np.float32)[:, None]
    i = np.arange(D_MODEL // 2, dtype=np.float32)[None, :]
    denom = np.power(10000.0, (2.0 * i) / D_MODEL)
    pe = np.zeros((MAX_LEN, D_MODEL), dtype=np.float32)
    pe[:, 0::2] = np.sin(pos / denom)
    pe[:, 1::2] = np.cos(pos / denom)
    return pe


def kernel(tokens, table):
    pe = jnp.asarray(_positional_encoding())
    tokT = tokens.reshape(NW, BT, MAX_LEN).transpose(0, 2, 1)
    out5 = _sc_embed(tokT, table, pe)
    out = out5.transpose(2, 4, 0, 1, 3).reshape(BATCH, MAX_LEN, D_MODEL)
    return out


# speedup vs baseline: 1.9694x; 1.0763x over previous
import functools

import jax
import jax.numpy as jnp
import numpy as np
from jax import lax
from jax.experimental import pallas as pl
from jax.experimental.pallas import tpu as pltpu
from jax.experimental.pallas import tpu_sc as plsc

VOCAB = 100000
MAX_LEN = 200
D_MODEL = 64
BATCH = 4096

NC, NS = 2, 16
NW = NC * NS
BT = BATCH // NW
NBUF = 4
SNB = 2
DT = D_MODEL // 8
NCH = D_MODEL // 16


@functools.partial(
    pl.kernel,
    out_type=jax.ShapeDtypeStruct((MAX_LEN, DT, NW, 8, BT), jnp.float32),
    mesh=plsc.VectorSubcoreMesh(core_axis_name="c", subcore_axis_name="s"),
    scratch_types=[
        pltpu.VMEM((MAX_LEN // 8, 8, BT), jnp.int32),
        pltpu.VMEM((MAX_LEN, D_MODEL), jnp.float32),
        pltpu.VMEM((NBUF, BT, D_MODEL), jnp.float32),
        pltpu.VMEM((SNB, DT, 8, BT), jnp.float32),
        pltpu.SemaphoreType.DMA((NBUF,)),
        pltpu.SemaphoreType.DMA((SNB,)),
    ],
    compiler_params=pltpu.CompilerParams(
        use_tc_tiling_on_sc=False, needs_layout_passes=False),
)
def _sc_embed(tok_hbm, table_hbm, pe_hbm, out_hbm,
              idx_v, pe_v, gbuf, sbuf, gsem, ssem):
    wid = lax.axis_index("s") * NC + lax.axis_index("c")

    pltpu.sync_copy(tok_hbm.at[:, wid], idx_v)
    pltpu.sync_copy(pe_hbm, pe_v)

    def fire_gather(l, b):
        pltpu.async_copy(table_hbm.at[idx_v.at[l // 8, l % 8]],
                         gbuf.at[b], gsem.at[b])

    def wait_gather(b):
        pltpu.make_async_copy(
            table_hbm.at[idx_v.at[0, 0]], gbuf.at[b], gsem.at[b]).wait()

    for b in range(NBUF):
        fire_gather(b, b)

    iota = lax.iota(jnp.int32, 16)
    blv = [iota + (q * 16) for q in range(BT // 16)]

    @pl.loop(0, MAX_LEN // NBUF)
    def _outer(o):
        for b in range(NBUF):
            l = o * NBUF + b
            sb = b % SNB
            wait_gather(b)
            if b >= SNB:
                pltpu.make_async_copy(
                    sbuf.at[sb], out_hbm.at[0, :, 0], ssem.at[sb]).wait()
            else:
                @pl.when(o > 0)
                def _():
                    pltpu.make_async_copy(
                        sbuf.at[sb], out_hbm.at[0, :, 0], ssem.at[sb]).wait()

            l_vec = lax.broadcast(l, (16,))

            @pl.loop(0, NCH)
            def _chunk(c):
                c16 = lax.broadcast(c * 16, (16,))
                c2 = lax.broadcast(c * 2, (16,))
                for t in range(16):
                    m = (iota + t) & 15
                    col = m + c16
                    pe_vec = plsc.load_gather(pe_v, [l_vec, col])
                    dtv = (m >> 3) + c2
                    dsv = m & 7
                    vs = [plsc.load_gather(gbuf.at[b], [blv[q], col])
                          for q in range(BT // 16)]
                    vs = [v + pe_vec for v in vs]
                    for q in range(BT // 16):
                        plsc.store_scatter(
                            sbuf.at[sb], [dtv, dsv, blv[q]], vs[q])

            pltpu.async_copy(sbuf.at[sb], out_hbm.at[l, :, wid], ssem.at[sb])

            @pl.when(o < MAX_LEN // NBUF - 1)
            def _():
                fire_gather(l + NBUF, b)

    for sb in range(SNB):
        pltpu.make_async_copy(
            sbuf.at[sb], out_hbm.at[0, :, 0], ssem.at[sb]).wait()


def _positional_encoding() -> np.ndarray:
    pos = np.arange(MAX_LEN, dtype=np.float32)[:, None]
    i = np.arange(D_MODEL // 2, dtype=np.float32)[None, :]
    denom = np.power(10000.0, (2.0 * i) / D_MODEL)
    pe = np.zeros((MAX_LEN, D_MODEL), dtype=np.float32)
    pe[:, 0::2] = np.sin(pos / denom)
    pe[:, 1::2] = np.cos(pos / denom)
    return pe


def kernel(tokens, table):
    pe = jnp.asarray(_positional_encoding())
    tok4 = tokens.T.reshape(MAX_LEN // 8, 8, NW, BT).transpose(0, 2, 1, 3)
    out5 = _sc_embed(tok4, table, pe)
    out = out5.transpose(2, 4, 0, 1, 3).reshape(BATCH, MAX_LEN, D_MODEL)
    return out
